# baseline (speedup 1.0000x reference)
import functools

import jax
import jax.numpy as jnp
from jax import lax
from jax.experimental import pallas as pl
from jax.experimental.pallas import tpu as pltpu
from jax.experimental.pallas import tpu_sc as plsc

_NC = 2
_NS = 16
_NW = _NC * _NS


def _make_lookup(B, V, D):
    b_per_w = B // _NW
    mesh = plsc.VectorSubcoreMesh(core_axis_name="c", subcore_axis_name="s")

    @functools.partial(
        pl.kernel,
        out_type=jax.ShapeDtypeStruct((B, D), jnp.float32),
        mesh=mesh,
        scratch_types=[
            pltpu.VMEM((b_per_w,), jnp.int32),
            pltpu.SemaphoreType.DMA,
        ],
    )
    def k(table_hbm, idx_hbm, out_hbm, idx_v, sem):
        cid = lax.axis_index("c")
        sid = lax.axis_index("s")
        wid = sid * _NC + cid
        base = wid * b_per_w
        pltpu.sync_copy(idx_hbm.at[pl.ds(base, b_per_w)], idx_v)

        def body(g, carry):
            vec = idx_v[pl.ds(g * 16, 16)]
            for j in range(16):
                row = vec[j]
                pltpu.async_copy(
                    table_hbm.at[pl.ds(row, 1)],
                    out_hbm.at[pl.ds(base + g * 16 + j, 1)],
                    sem,
                )
            return carry

        lax.fori_loop(0, b_per_w // 16, body, 0)

        def drain(r, carry):
            pltpu.make_async_copy(
                table_hbm.at[pl.ds(0, 1)],
                out_hbm.at[pl.ds(base, 1)],
                sem,
            ).wait()
            return carry

        lax.fori_loop(0, b_per_w, drain, 0)

    return k


def kernel(position_ids, table):
    batch, seq = position_ids.shape
    V, D = table.shape
    flat_ids = position_ids.reshape(-1).astype(jnp.int32)
    out = _make_lookup(flat_ids.shape[0], V, D)(table, flat_ids)
    return out.reshape(batch, seq, D)


# speedup vs baseline: 36.3539x; 36.3539x over previous
import functools

import jax
import jax.numpy as jnp
from jax import lax
from jax.experimental import pallas as pl
from jax.experimental.pallas import tpu as pltpu
from jax.experimental.pallas import tpu_sc as plsc

_NC = 2
_NS = 16
_NW = _NC * _NS
_C = 16
_NBUF = 4


def _make_lookup(B, V, D):
    b_per_w = B // _NW
    n_chunks = b_per_w // _C
    assert n_chunks % _NBUF == 0
    mesh = plsc.VectorSubcoreMesh(core_axis_name="c", subcore_axis_name="s")

    @functools.partial(
        pl.kernel,
        out_type=jax.ShapeDtypeStruct((B, D), jnp.float32),
        mesh=mesh,
        compiler_params=pltpu.CompilerParams(disable_semaphore_checks=True, skip_device_barrier=True),
        scratch_types=[
            pltpu.VMEM((b_per_w,), jnp.int32),
            [pltpu.VMEM((_C, D), jnp.float32) for _ in range(_NBUF)],
            [pltpu.SemaphoreType.DMA for _ in range(_NBUF)],
            [pltpu.SemaphoreType.DMA for _ in range(_NBUF)],
        ],
    )
    def k(table_hbm, idx_hbm, out_hbm, idx_v, bufs, gsems, wsems):
        wid = lax.axis_index("s") * _NC + lax.axis_index("c")
        base = wid * b_per_w
        pltpu.sync_copy(idx_hbm.at[pl.ds(base, b_per_w)], idx_v)

        def start_gather(c, b):
            pltpu.async_copy(
                table_hbm.at[idx_v.at[pl.ds(c * _C, _C)]], bufs[b], gsems[b]
            )

        def wait_gather(b):
            pltpu.make_async_copy(
                table_hbm.at[pl.ds(0, _C)], bufs[b], gsems[b]
            ).wait()

        def start_write(c, b):
            pltpu.async_copy(
                bufs[b], out_hbm.at[pl.ds(base + c * _C, _C)], wsems[b]
            )

        def wait_write(b):
            pltpu.make_async_copy(
                bufs[b], out_hbm.at[pl.ds(base, _C)], wsems[b]
            ).wait()

        start_gather(0, 0)
        start_gather(1, 1)

        def body(g4, carry):
            for b in range(_NBUF):
                c = g4 * _NBUF + b
                bn = (b + 2) % _NBUF

                @pl.when(c >= 2)
                def _():
                    wait_write(bn)

                @pl.when(c + 2 < n_chunks)
                def _():
                    start_gather(c + 2, bn)

                wait_gather(b)
                start_write(c, b)
            return carry

        lax.fori_loop(0, n_chunks // _NBUF, body, 0)
        wait_write((n_chunks - 2) % _NBUF)
        wait_write((n_chunks - 1) % _NBUF)

    return k


def kernel(position_ids, table):
    batch, seq = position_ids.shape
    V, D = table.shape
    flat_ids = position_ids.reshape(-1).astype(jnp.int32)
    out = _make_lookup(flat_ids.shape[0], V, D)(table, flat_ids)
    return out.reshape(batch, seq, D)


# speedup vs baseline: 36.6860x; 1.0091x over previous
import functools

import jax
import jax.numpy as jnp
from jax import lax
from jax.experimental import pallas as pl
from jax.experimental.pallas import tpu as pltpu
from jax.experimental.pallas import tpu_sc as plsc

_NC = 2
_NS = 16
_NW = _NC * _NS
_C = 8
_NBUF = 8
_LOOK = 4


def _make_lookup(B, V, D):
    b_per_w = B // _NW
    n_chunks = b_per_w // _C
    assert n_chunks % _NBUF == 0
    mesh = plsc.VectorSubcoreMesh(core_axis_name="c", subcore_axis_name="s")

    @functools.partial(
        pl.kernel,
        out_type=jax.ShapeDtypeStruct((B, D), jnp.float32),
        mesh=mesh,
        compiler_params=pltpu.CompilerParams(disable_semaphore_checks=True, skip_device_barrier=True),
        scratch_types=[
            pltpu.VMEM((b_per_w,), jnp.int32),
            [pltpu.VMEM((_C, D), jnp.float32) for _ in range(_NBUF)],
            [pltpu.SemaphoreType.DMA for _ in range(_NBUF)],
            [pltpu.SemaphoreType.DMA for _ in range(_NBUF)],
        ],
    )
    def k(table_hbm, idx_hbm, out_hbm, idx_v, bufs, gsems, wsems):
        wid = lax.axis_index("s") * _NC + lax.axis_index("c")
        base = wid * b_per_w
        pltpu.sync_copy(idx_hbm.at[pl.ds(base, b_per_w)], idx_v)

        def start_gather(c, b):
            pltpu.async_copy(
                table_hbm.at[idx_v.at[pl.ds(c * _C, _C)]], bufs[b], gsems[b]
            )

        def wait_gather(b):
            pltpu.make_async_copy(
                table_hbm.at[pl.ds(0, _C)], bufs[b], gsems[b]
            ).wait()

        def start_write(c, b):
            pltpu.async_copy(
                bufs[b], out_hbm.at[pl.ds(base + c * _C, _C)], wsems[b]
            )

        def wait_write(b):
            pltpu.make_async_copy(
                bufs[b], out_hbm.at[pl.ds(base, _C)], wsems[b]
            ).wait()

        for p in range(_LOOK):
            start_gather(p, p)

        def body(g4, carry):
            for b in range(_NBUF):
                c = g4 * _NBUF + b
                bn = (b + _LOOK) % _NBUF

                @pl.when(c >= _LOOK)
                def _():
                    wait_write(bn)

                @pl.when(c + _LOOK < n_chunks)
                def _():
                    start_gather(c + _LOOK, bn)

                wait_gather(b)
                start_write(c, b)
            return carry

        lax.fori_loop(0, n_chunks // _NBUF, body, 0)
        for p in range(_LOOK, 0, -1):
            wait_write((n_chunks - p) % _NBUF)

    return k


def kernel(position_ids, table):
    batch, seq = position_ids.shape
    V, D = table.shape
    flat_ids = position_ids.reshape(-1).astype(jnp.int32)
    out = _make_lookup(flat_ids.shape[0], V, D)(table, flat_ids)
    return out.reshape(batch, seq, D)


# speedup vs baseline: 36.6885x; 1.0001x over previous
import functools

import jax
import jax.numpy as jnp
from jax import lax
from jax.experimental import pallas as pl
from jax.experimental.pallas import tpu as pltpu
from jax.experimental.pallas import tpu_sc as plsc

_NC = 2
_NS = 16
_NW = _NC * _NS
_C = 8
_NBUF = 8
_LOOK = 4


def _make_lookup(B, V, D):
    b_per_w = B // _NW
    n_chunks = b_per_w // _C
    assert n_chunks % _NBUF == 0
    mesh = plsc.VectorSubcoreMesh(core_axis_name="c", subcore_axis_name="s")

    @functools.partial(
        pl.kernel,
        out_type=jax.ShapeDtypeStruct((B, D), jnp.float32),
        mesh=mesh,
        scratch_types=[
            pltpu.VMEM((b_per_w,), jnp.int32),
            [pltpu.VMEM((_C, D), jnp.float32) for _ in range(_NBUF)],
            [pltpu.SemaphoreType.DMA for _ in range(_NBUF)],
            [pltpu.SemaphoreType.DMA for _ in range(_NBUF)],
        ],
    )
    def k(table_hbm, idx_hbm, out_hbm, idx_v, bufs, gsems, wsems):
        wid = lax.axis_index("s") * _NC + lax.axis_index("c")
        base = wid * b_per_w
        pltpu.sync_copy(idx_hbm.at[pl.ds(base, b_per_w)], idx_v)

        def start_gather(c, b):
            pltpu.async_copy(
                table_hbm.at[idx_v.at[pl.ds(c * _C, _C)]], bufs[b], gsems[b]
            )

        def wait_gather(b):
            pltpu.make_async_copy(
                table_hbm.at[pl.ds(0, _C)], bufs[b], gsems[b]
            ).wait()

        def start_write(c, b):
            pltpu.async_copy(
                bufs[b], out_hbm.at[pl.ds(base + c * _C, _C)], wsems[b]
            )

        def wait_write(b):
            pltpu.make_async_copy(
                bufs[b], out_hbm.at[pl.ds(base, _C)], wsems[b]
            ).wait()

        for p in range(_LOOK):
            start_gather(p, p)

        def body(g4, carry):
            for b in range(_NBUF):
                c = g4 * _NBUF + b
                bn = (b + _LOOK) % _NBUF

                @pl.when(c >= _LOOK)
                def _():
                    wait_write(bn)

                @pl.when(c + _LOOK < n_chunks)
                def _():
                    start_gather(c + _LOOK, bn)

                wait_gather(b)
                start_write(c, b)
            return carry

        lax.fori_loop(0, n_chunks // _NBUF, body, 0)
        for p in range(_LOOK, 0, -1):
            wait_write((n_chunks - p) % _NBUF)

    return k


def kernel(position_ids, table):
    batch, seq = position_ids.shape
    V, D = table.shape
    flat_ids = position_ids.reshape(-1).astype(jnp.int32)
    out = _make_lookup(flat_ids.shape[0], V, D)(table, flat_ids)
    return out.reshape(batch, seq, D)
